# baseline (speedup 1.0000x reference)
import functools

import jax
import jax.numpy as jnp
from jax import lax
from jax.experimental import pallas as pl
from jax.experimental.pallas import tpu as pltpu
from jax.experimental.pallas import tpu_sc as plsc

_D = 128
_N_EDGES = 320000
_N_NODES = 10000
_N_NODES_PAD = 10240
_NODES_PER_TILE = 640
_EDGES_PER_TILE = _N_EDGES // 32
_GCHUNK = 128
_N_NODES_TBL = 100000
_N_RELS = 400000


_CHUNK = 2000
_RING = 24


def _rowsums_body(ent_hbm, rel_hbm, se_ref, sr_ref, ring, sems):
    ones = jnp.ones((1, _D), jnp.float32)
    dn = (((1,), (1,)), ((), ()))

    for tbl, out_ref in ((ent_hbm, se_ref), (rel_hbm, sr_ref)):
        nchunk = tbl.shape[0] // _CHUNK

        def issue(c, slot, tbl=tbl):
            off = pl.multiple_of(c * _CHUNK, _CHUNK)
            pltpu.make_async_copy(
                tbl.at[pl.ds(off, _CHUNK), :], ring.at[slot],
                sems.at[slot]).start()

        for k in range(min(_RING, nchunk)):
            issue(k, k)

        def step(i, carry, tbl=tbl, out_ref=out_ref, nchunk=nchunk):
            slot = lax.rem(i, _RING)
            off = pl.multiple_of(i * _CHUNK, _CHUNK)
            pltpu.make_async_copy(
                tbl.at[pl.ds(off, _CHUNK), :], ring.at[slot],
                sems.at[slot]).wait()
            out_ref[pl.ds(i, 1), :] = jax.lax.dot_general(
                ones, ring[slot], dn, preferred_element_type=jnp.float32)

            @pl.when(i + _RING < nchunk)
            def _():
                issue(i + _RING, slot)

            return carry

        lax.fori_loop(0, nchunk, step, 0)


def _rowsums(ent_table, rel_table):
    se, sr = pl.pallas_call(
        _rowsums_body,
        in_specs=[
            pl.BlockSpec(memory_space=pl.ANY),
            pl.BlockSpec(memory_space=pl.ANY),
        ],
        out_shape=[
            jax.ShapeDtypeStruct((_N_NODES_TBL // _CHUNK, _CHUNK), jnp.float32),
            jax.ShapeDtypeStruct((_N_RELS // _CHUNK, _CHUNK), jnp.float32),
        ],
        scratch_shapes=[
            pltpu.VMEM((_RING, _CHUNK, _D), jnp.float32),
            pltpu.SemaphoreType.DMA((_RING,)),
        ],
    )(ent_table, rel_table)
    return se.reshape(_N_NODES_TBL), sr.reshape(_N_RELS)


def _make_sc_combine():
    mesh = plsc.VectorSubcoreMesh(core_axis_name="c", subcore_axis_name="s")

    @functools.partial(
        pl.kernel,
        out_type=jax.ShapeDtypeStruct((_N_EDGES,), jnp.float32),
        mesh=mesh,
        compiler_params=pltpu.CompilerParams(needs_layout_passes=False),
        scratch_types=[
            pltpu.VMEM((_NODES_PER_TILE,), jnp.int32),
            pltpu.VMEM((_NODES_PER_TILE,), jnp.float32),
            pltpu.VMEM_SHARED((_N_NODES_PAD,), jnp.float32),
            pltpu.VMEM((_N_NODES_PAD,), jnp.float32),
            pltpu.VMEM((_EDGES_PER_TILE,), jnp.int32),
            pltpu.VMEM((_EDGES_PER_TILE,), jnp.int32),
            pltpu.VMEM((_EDGES_PER_TILE,), jnp.int32),
            pltpu.VMEM((_EDGES_PER_TILE,), jnp.float32),
            pltpu.VMEM((_EDGES_PER_TILE,), jnp.float32),
            pltpu.SemaphoreType.DMA,
            pltpu.SemaphoreType.DMA,
            pltpu.SemaphoreType.DMA,
        ],
    )
    def sc_combine(se_hbm, sr_hbm, nid_hbm, edge_hbm, rel_hbm,
                   out_hbm, nid_v, nsum_v, sh_shared, sh_v, src_v, dst_v,
                   rel_v, r_v, out_v, sem1, sem2, sem3):
        cid = lax.axis_index("c")
        sid = lax.axis_index("s")
        wid = sid * 2 + cid

        ebase = pl.multiple_of(wid * _EDGES_PER_TILE, 8)
        h_src = pltpu.async_copy(
            edge_hbm.at[pl.ds(ebase, _EDGES_PER_TILE)], src_v, sem3)
        dbase = pl.multiple_of(_N_EDGES + wid * _EDGES_PER_TILE, 8)
        h_dst = pltpu.async_copy(
            edge_hbm.at[pl.ds(dbase, _EDGES_PER_TILE)], dst_v, sem3)
        h_rel = pltpu.async_copy(
            rel_hbm.at[pl.ds(ebase, _EDGES_PER_TILE)], rel_v, sem3)

        nbase = pl.multiple_of(sid * _NODES_PER_TILE, 8)
        pltpu.sync_copy(nid_hbm.at[pl.ds(nbase, _NODES_PER_TILE)], nid_v)
        ph1 = []
        for j in range(_NODES_PER_TILE // _GCHUNK):
            ph1.append(pltpu.async_copy(
                se_hbm.at[nid_v.at[pl.ds(j * _GCHUNK, _GCHUNK)]],
                nsum_v.at[pl.ds(j * _GCHUNK, _GCHUNK)], sem1))
        for h in ph1:
            h.wait()
        pltpu.sync_copy(nsum_v, sh_shared.at[pl.ds(nbase, _NODES_PER_TILE)])
        plsc.subcore_barrier()
        pltpu.sync_copy(sh_shared, sh_v)

        h_src.wait()
        h_dst.wait()
        h_rel.wait()

        handles = []
        nfull = _EDGES_PER_TILE // _GCHUNK
        for j in range(nfull):
            handles.append(pltpu.async_copy(
                sr_hbm.at[rel_v.at[pl.ds(j * _GCHUNK, _GCHUNK)]],
                r_v.at[pl.ds(j * _GCHUNK, _GCHUNK)], sem2))
            if len(handles) > 13:
                handles.pop(0).wait()
        rem = _EDGES_PER_TILE - nfull * _GCHUNK
        if rem:
            handles.append(pltpu.async_copy(
                sr_hbm.at[rel_v.at[pl.ds(nfull * _GCHUNK, rem)]],
                r_v.at[pl.ds(nfull * _GCHUNK, rem)], sem2))
        for h in handles:
            h.wait()

        @plsc.parallel_loop(0, _EDGES_PER_TILE, step=16, unroll=4)
        def _(o):
            s16 = src_v[pl.ds(o, 16)]
            d16 = dst_v[pl.ds(o, 16)]
            hvec = plsc.load_gather(sh_v, [s16])
            tvec = plsc.load_gather(sh_v, [d16])
            out_v[pl.ds(o, 16)] = hvec + r_v[pl.ds(o, 16)] - tvec

        pltpu.sync_copy(out_v, out_hbm.at[pl.ds(ebase, _EDGES_PER_TILE)])

    return sc_combine


_sc_combine = _make_sc_combine()


def kernel(ent_table, rel_table, node_ids, edge_index, edge_rel_ids):
    se, sr = _rowsums(ent_table, rel_table)
    nid_pad = jnp.concatenate(
        [node_ids, jnp.zeros((_N_NODES_PAD - _N_NODES,), jnp.int32)])
    return _sc_combine(se, sr, nid_pad, edge_index.reshape(-1), edge_rel_ids)


# speedup vs baseline: 1.0162x; 1.0162x over previous
import functools

import jax
import jax.numpy as jnp
from jax import lax
from jax.experimental import pallas as pl
from jax.experimental.pallas import tpu as pltpu
from jax.experimental.pallas import tpu_sc as plsc

_D = 128
_N_EDGES = 320000
_N_NODES = 10000
_N_NODES_PAD = 10240
_NODES_PER_TILE = 640
_EDGES_PER_TILE = _N_EDGES // 32
_GCHUNK = 128
_N_NODES_TBL = 100000
_N_RELS = 400000


_CHUNK = 4000
_RING = 12


def _rowsums_body(ent_hbm, rel_hbm, se_ref, sr_ref, ring, sems):
    ones = jnp.ones((1, _D), jnp.float32)
    dn = (((1,), (1,)), ((), ()))

    for tbl, out_ref in ((ent_hbm, se_ref), (rel_hbm, sr_ref)):
        nchunk = tbl.shape[0] // _CHUNK

        def issue(c, slot, tbl=tbl):
            off = pl.multiple_of(c * _CHUNK, _CHUNK)
            pltpu.make_async_copy(
                tbl.at[pl.ds(off, _CHUNK), :], ring.at[slot],
                sems.at[slot]).start()

        for k in range(min(_RING, nchunk)):
            issue(k, k)

        def step(i, carry, tbl=tbl, out_ref=out_ref, nchunk=nchunk):
            slot = lax.rem(i, _RING)
            off = pl.multiple_of(i * _CHUNK, _CHUNK)
            pltpu.make_async_copy(
                tbl.at[pl.ds(off, _CHUNK), :], ring.at[slot],
                sems.at[slot]).wait()
            out_ref[pl.ds(i, 1), :] = jax.lax.dot_general(
                ones, ring[slot], dn, preferred_element_type=jnp.float32)

            @pl.when(i + _RING < nchunk)
            def _():
                issue(i + _RING, slot)

            return carry

        lax.fori_loop(0, nchunk, step, 0)


def _rowsums(ent_table, rel_table):
    se, sr = pl.pallas_call(
        _rowsums_body,
        in_specs=[
            pl.BlockSpec(memory_space=pl.ANY),
            pl.BlockSpec(memory_space=pl.ANY),
        ],
        out_shape=[
            jax.ShapeDtypeStruct((_N_NODES_TBL // _CHUNK, _CHUNK), jnp.float32),
            jax.ShapeDtypeStruct((_N_RELS // _CHUNK, _CHUNK), jnp.float32),
        ],
        scratch_shapes=[
            pltpu.VMEM((_RING, _CHUNK, _D), jnp.float32),
            pltpu.SemaphoreType.DMA((_RING,)),
        ],
    )(ent_table, rel_table)
    return se.reshape(_N_NODES_TBL), sr.reshape(_N_RELS)


def _make_sc_combine():
    mesh = plsc.VectorSubcoreMesh(core_axis_name="c", subcore_axis_name="s")

    @functools.partial(
        pl.kernel,
        out_type=jax.ShapeDtypeStruct((_N_EDGES,), jnp.float32),
        mesh=mesh,
        compiler_params=pltpu.CompilerParams(needs_layout_passes=False),
        scratch_types=[
            pltpu.VMEM((_NODES_PER_TILE,), jnp.int32),
            pltpu.VMEM((_NODES_PER_TILE,), jnp.float32),
            pltpu.VMEM_SHARED((_N_NODES_PAD,), jnp.float32),
            pltpu.VMEM((_N_NODES_PAD,), jnp.float32),
            pltpu.VMEM((_EDGES_PER_TILE,), jnp.int32),
            pltpu.VMEM((_EDGES_PER_TILE,), jnp.int32),
            pltpu.VMEM((_EDGES_PER_TILE,), jnp.int32),
            pltpu.VMEM((_EDGES_PER_TILE,), jnp.float32),
            pltpu.VMEM((_EDGES_PER_TILE,), jnp.float32),
            pltpu.SemaphoreType.DMA,
            pltpu.SemaphoreType.DMA,
            pltpu.SemaphoreType.DMA,
        ],
    )
    def sc_combine(se_hbm, sr_hbm, nid_hbm, edge_hbm, rel_hbm,
                   out_hbm, nid_v, nsum_v, sh_shared, sh_v, src_v, dst_v,
                   rel_v, r_v, out_v, sem1, sem2, sem3):
        cid = lax.axis_index("c")
        sid = lax.axis_index("s")
        wid = sid * 2 + cid

        ebase = pl.multiple_of(wid * _EDGES_PER_TILE, 8)
        h_src = pltpu.async_copy(
            edge_hbm.at[pl.ds(ebase, _EDGES_PER_TILE)], src_v, sem3)
        dbase = pl.multiple_of(_N_EDGES + wid * _EDGES_PER_TILE, 8)
        h_dst = pltpu.async_copy(
            edge_hbm.at[pl.ds(dbase, _EDGES_PER_TILE)], dst_v, sem3)
        h_rel = pltpu.async_copy(
            rel_hbm.at[pl.ds(ebase, _EDGES_PER_TILE)], rel_v, sem3)

        nbase = pl.multiple_of(sid * _NODES_PER_TILE, 8)
        pltpu.sync_copy(nid_hbm.at[pl.ds(nbase, _NODES_PER_TILE)], nid_v)
        ph1 = []
        for j in range(_NODES_PER_TILE // _GCHUNK):
            ph1.append(pltpu.async_copy(
                se_hbm.at[nid_v.at[pl.ds(j * _GCHUNK, _GCHUNK)]],
                nsum_v.at[pl.ds(j * _GCHUNK, _GCHUNK)], sem1))

        h_rel.wait()
        handles = []
        nfull = _EDGES_PER_TILE // _GCHUNK
        for j in range(13):
            handles.append(pltpu.async_copy(
                sr_hbm.at[rel_v.at[pl.ds(j * _GCHUNK, _GCHUNK)]],
                r_v.at[pl.ds(j * _GCHUNK, _GCHUNK)], sem2))

        for h in ph1:
            h.wait()
        pltpu.sync_copy(nsum_v, sh_shared.at[pl.ds(nbase, _NODES_PER_TILE)])
        plsc.subcore_barrier()
        pltpu.sync_copy(sh_shared, sh_v)

        for j in range(13, nfull):
            handles.append(pltpu.async_copy(
                sr_hbm.at[rel_v.at[pl.ds(j * _GCHUNK, _GCHUNK)]],
                r_v.at[pl.ds(j * _GCHUNK, _GCHUNK)], sem2))
            if len(handles) > 13:
                handles.pop(0).wait()
        rem = _EDGES_PER_TILE - nfull * _GCHUNK
        if rem:
            handles.append(pltpu.async_copy(
                sr_hbm.at[rel_v.at[pl.ds(nfull * _GCHUNK, rem)]],
                r_v.at[pl.ds(nfull * _GCHUNK, rem)], sem2))
        h_src.wait()
        h_dst.wait()
        for h in handles:
            h.wait()

        @plsc.parallel_loop(0, _EDGES_PER_TILE, step=16, unroll=4)
        def _(o):
            s16 = src_v[pl.ds(o, 16)]
            d16 = dst_v[pl.ds(o, 16)]
            hvec = plsc.load_gather(sh_v, [s16])
            tvec = plsc.load_gather(sh_v, [d16])
            out_v[pl.ds(o, 16)] = hvec + r_v[pl.ds(o, 16)] - tvec

        pltpu.sync_copy(out_v, out_hbm.at[pl.ds(ebase, _EDGES_PER_TILE)])

    return sc_combine


_sc_combine = _make_sc_combine()


def kernel(ent_table, rel_table, node_ids, edge_index, edge_rel_ids):
    se, sr = _rowsums(ent_table, rel_table)
    nid_pad = jnp.concatenate(
        [node_ids, jnp.zeros((_N_NODES_PAD - _N_NODES,), jnp.int32)])
    return _sc_combine(se, sr, nid_pad, edge_index.reshape(-1), edge_rel_ids)
